# baseline (speedup 1.0000x reference)
import functools

import jax
import jax.numpy as jnp
from jax import lax
from jax.experimental import pallas as pl
from jax.experimental.pallas import tpu as pltpu
from jax.experimental.pallas import tpu_sc as plsc

_BATCH = 16384
_NC = 2
_NS = 16
_NW = _NC * _NS
_BPW = _BATCH // _NW
_NCH = 8
_CH = _BPW // _NCH


def _gather_body(sid_hbm, table_hbm, out_hbm, idx_v, val_v,
                 sem_i, sem_g, sem_s):
    wid = lax.axis_index("s") * _NC + lax.axis_index("c")
    base = wid * _BPW
    loads = [
        pltpu.async_copy(sid_hbm.at[pl.ds(base + j * _CH, _CH)],
                         idx_v.at[j], sem_i[j])
        for j in range(_NCH)
    ]
    gathers = []
    for j in range(_NCH):
        loads[j].wait()
        gathers.append(
            pltpu.async_copy(table_hbm.at[idx_v.at[j]], val_v.at[j],
                             sem_g[j]))
    stores = []
    for j in range(_NCH):
        gathers[j].wait()
        stores.append(
            pltpu.async_copy(val_v.at[j],
                             out_hbm.at[pl.ds(base + j * _CH, _CH)],
                             sem_s[j]))
    for j in range(_NCH):
        stores[j].wait()


@jax.jit
def kernel(sid, table):
    mesh = plsc.VectorSubcoreMesh(core_axis_name="c", subcore_axis_name="s")
    run = pl.kernel(
        _gather_body,
        mesh=mesh,
        out_type=jax.ShapeDtypeStruct((_BATCH,), jnp.float32),
        scratch_types=[
            pltpu.VMEM((_NCH, _CH), jnp.int32),
            pltpu.VMEM((_NCH, _CH), jnp.float32),
            [pltpu.SemaphoreType.DMA] * _NCH,
            [pltpu.SemaphoreType.DMA] * _NCH,
            [pltpu.SemaphoreType.DMA] * _NCH,
        ],
    )
    return run(sid, table)


# speedup vs baseline: 1.0353x; 1.0353x over previous
import functools

import jax
import jax.numpy as jnp
from jax import lax
from jax.experimental import pallas as pl
from jax.experimental.pallas import tpu as pltpu
from jax.experimental.pallas import tpu_sc as plsc

_BATCH = 16384
_NC = 2
_NS = 16
_NW = _NC * _NS
_BPW = _BATCH // _NW
_NCH = 2
_CH = _BPW // _NCH


def _gather_body(sid_hbm, table_hbm, out_hbm, idx_v, val_v,
                 sem_i, sem_g, sem_s):
    wid = lax.axis_index("s") * _NC + lax.axis_index("c")
    base = wid * _BPW
    pltpu.sync_copy(sid_hbm.at[pl.ds(base, _BPW)], idx_v)
    gathers = []
    for j in range(_NCH):
        gathers.append(
            pltpu.async_copy(table_hbm.at[idx_v.at[pl.ds(j * _CH, _CH)]],
                             val_v.at[pl.ds(j * _CH, _CH)], sem_g[j]))
    stores = []
    for j in range(_NCH):
        gathers[j].wait()
        stores.append(
            pltpu.async_copy(val_v.at[pl.ds(j * _CH, _CH)],
                             out_hbm.at[pl.ds(base + j * _CH, _CH)],
                             sem_s[j]))
    for j in range(_NCH):
        stores[j].wait()


@jax.jit
def kernel(sid, table):
    mesh = plsc.VectorSubcoreMesh(core_axis_name="c", subcore_axis_name="s")
    run = pl.kernel(
        _gather_body,
        mesh=mesh,
        out_type=jax.ShapeDtypeStruct((_BATCH,), jnp.float32),
        scratch_types=[
            pltpu.VMEM((_BPW,), jnp.int32),
            pltpu.VMEM((_BPW,), jnp.float32),
            [pltpu.SemaphoreType.DMA] * _NCH,
            [pltpu.SemaphoreType.DMA] * _NCH,
            [pltpu.SemaphoreType.DMA] * _NCH,
        ],
    )
    return run(sid, table)
